# baseline (speedup 1.0000x reference)
import functools

import jax
import jax.numpy as jnp
from jax import lax
from jax.experimental import pallas as pl
from jax.experimental.pallas import tpu as pltpu
from jax.experimental.pallas import tpu_sc as plsc

_NUM_CORES = 2
_NUM_SUBCORES = 16
_NW = _NUM_CORES * _NUM_SUBCORES
_LANES = 16
_UNROLL = 5
_NBUF = 4


def _make_pool_kernel(B, Lseq, D):
    rows_per_w = B // _NW
    CR = 2
    chunk_len = CR * Lseq
    n_chunks = rows_per_w // CR
    nsub = D // _LANES
    scale = 1.0 / Lseq
    mesh = plsc.VectorSubcoreMesh(
        core_axis_name="c", subcore_axis_name="s",
        num_cores=_NUM_CORES, num_subcores=_NUM_SUBCORES)

    @functools.partial(
        pl.kernel,
        out_type=jax.ShapeDtypeStruct((B, D), jnp.float32),
        mesh=mesh,
        scratch_types=[
            pltpu.VMEM((n_chunks, chunk_len), jnp.int32),
            pltpu.VMEM((_NBUF, chunk_len, D), jnp.float32),
            pltpu.VMEM((rows_per_w, D), jnp.float32),
            [pltpu.SemaphoreType.DMA] * _NBUF,
            pltpu.SemaphoreType.DMA,
        ],
    )
    def pool(tok_hbm, emb_hbm, out_hbm, idx_v, rows_v, pooled_v, gsem,
             osem):
        wid = lax.axis_index("s") * _NUM_CORES + lax.axis_index("c")
        base_row = wid * rows_per_w
        pltpu.sync_copy(tok_hbm.at[wid], idx_v)

        def start(chunk, b):
            pltpu.async_copy(emb_hbm.at[idx_v.at[chunk]], rows_v.at[b],
                             gsem[b])

        def wait(chunk, b):
            pltpu.make_async_copy(
                emb_hbm.at[idx_v.at[chunk]], rows_v.at[b], gsem[b]).wait()

        def accumulate(chunk, b):
            for r in range(CR):
                def body(t, accs, r=r):
                    base = r * Lseq + t * _UNROLL
                    for u in range(_UNROLL):
                        accs = tuple(
                            accs[c] + rows_v[b, base + u,
                                             pl.ds(c * _LANES, _LANES)]
                            for c in range(nsub))
                    return accs
                accs = lax.fori_loop(
                    0, Lseq // _UNROLL, body,
                    tuple(jnp.zeros((_LANES,), jnp.float32)
                          for _ in range(nsub)))
                row = chunk * CR + r
                for c in range(nsub):
                    pooled_v[row, pl.ds(c * _LANES, _LANES)] = accs[c] * scale

        look = _NBUF - 1
        for p in range(look):
            start(p, p)

        def visit(k, b, cool):
            wait(k, b)
            if cool:
                start(k + look, (b + look) % _NBUF)
            accumulate(k, b)

        ngrp = (n_chunks - look) // _NBUF

        def grp(i, carry):
            k = _NBUF * i
            for q in range(_NBUF):
                visit(k + q, q, cool=True)
            return carry
        half_g = ngrp // 2
        half_rows = half_g * _NBUF * CR
        lax.fori_loop(0, half_g, grp, 0)
        out_half = pltpu.async_copy(
            pooled_v.at[pl.ds(0, half_rows)],
            out_hbm.at[pl.ds(base_row, half_rows)], osem)
        lax.fori_loop(half_g, ngrp, grp, 0)

        for kk in range(_NBUF * ngrp, n_chunks):
            visit(kk, kk % _NBUF, cool=kk + look < n_chunks)

        pltpu.sync_copy(
            pooled_v.at[pl.ds(half_rows, rows_per_w - half_rows)],
            out_hbm.at[pl.ds(base_row + half_rows,
                             rows_per_w - half_rows)])
        out_half.wait()

    return pool


def _project(pooled, W, b):
    B, D = pooled.shape
    M = W.shape[0]
    BLK = 4096

    def mm(x_ref, w_ref, b_ref, o_ref):
        o_ref[...] = lax.dot_general(
            x_ref[...], w_ref[...], (((1,), (1,)), ((), ())),
            preferred_element_type=jnp.float32) + b_ref[...]

    return pl.pallas_call(
        mm,
        grid=(B // BLK,),
        in_specs=[
            pl.BlockSpec((BLK, D), lambda i: (i, 0)),
            pl.BlockSpec((M, D), lambda i: (0, 0)),
            pl.BlockSpec((1, M), lambda i: (0, 0)),
        ],
        out_specs=pl.BlockSpec((BLK, M), lambda i: (i, 0)),
        out_shape=jax.ShapeDtypeStruct((B, M), jnp.float32),
    )(pooled, W, b.reshape(1, M))


def kernel(token_ids, emb, W, b):
    B, Lseq = token_ids.shape
    idx_per_w = (B // _NW) * Lseq
    chunk = 2 * Lseq
    tok = token_ids.astype(jnp.int32).reshape(
        _NW, idx_per_w // chunk, chunk)
    pooled = _make_pool_kernel(B, Lseq, emb.shape[1])(tok, emb)
    out = _project(pooled, W, b)
    return out[:, None, :]


# speedup vs baseline: 1.0260x; 1.0260x over previous
import functools

import jax
import jax.numpy as jnp
from jax import lax
from jax.experimental import pallas as pl
from jax.experimental.pallas import tpu as pltpu
from jax.experimental.pallas import tpu_sc as plsc

_NUM_CORES = 2
_NUM_SUBCORES = 16
_NW = _NUM_CORES * _NUM_SUBCORES
_LANES = 16
_UNROLL = 5
_NBUF = 4


def _make_pool_kernel(B, Lseq, D):
    rows_per_w = B // _NW
    CR = 2
    chunk_len = CR * Lseq
    n_chunks = rows_per_w // CR
    nsub = D // _LANES
    scale = 1.0 / Lseq
    mesh = plsc.VectorSubcoreMesh(
        core_axis_name="c", subcore_axis_name="s",
        num_cores=_NUM_CORES, num_subcores=_NUM_SUBCORES)

    @functools.partial(
        pl.kernel,
        out_type=jax.ShapeDtypeStruct((B, D), jnp.float32),
        mesh=mesh,
        scratch_types=[
            pltpu.VMEM((n_chunks, chunk_len), jnp.int32),
            pltpu.VMEM((_NBUF, chunk_len, D), jnp.float32),
            pltpu.VMEM((rows_per_w, D), jnp.float32),
            [pltpu.SemaphoreType.DMA] * _NBUF,
        ],
    )
    def pool(tok_hbm, emb_hbm, out_hbm, idx_v, rows_v, pooled_v, gsem):
        wid = lax.axis_index("s") * _NUM_CORES + lax.axis_index("c")
        base_row = wid * rows_per_w
        pltpu.sync_copy(tok_hbm.at[wid], idx_v)

        def start(chunk, b):
            pltpu.async_copy(emb_hbm.at[idx_v.at[chunk]], rows_v.at[b],
                             gsem[b])

        def wait(chunk, b):
            pltpu.make_async_copy(
                emb_hbm.at[idx_v.at[chunk]], rows_v.at[b], gsem[b]).wait()

        def accumulate(chunk, b):
            for r in range(CR):
                def body(t, accs, r=r):
                    base = r * Lseq + t * _UNROLL
                    for u in range(_UNROLL):
                        accs = tuple(
                            accs[c] + rows_v[b, base + u,
                                             pl.ds(c * _LANES, _LANES)]
                            for c in range(nsub))
                    return accs
                accs = lax.fori_loop(
                    0, Lseq // _UNROLL, body,
                    tuple(jnp.zeros((_LANES,), jnp.float32)
                          for _ in range(nsub)))
                row = chunk * CR + r
                for c in range(nsub):
                    pooled_v[row, pl.ds(c * _LANES, _LANES)] = accs[c] * scale

        look = _NBUF - 1
        for p in range(look):
            start(p, p)

        def visit(k, b, cool):
            wait(k, b)
            if cool:
                start(k + look, (b + look) % _NBUF)
            accumulate(k, b)

        ngrp = (n_chunks - look) // _NBUF

        def grp(i, carry):
            k = _NBUF * i
            for q in range(_NBUF):
                visit(k + q, q, cool=True)
            return carry
        lax.fori_loop(0, ngrp, grp, 0)

        for kk in range(_NBUF * ngrp, n_chunks):
            visit(kk, kk % _NBUF, cool=kk + look < n_chunks)

        pltpu.sync_copy(pooled_v, out_hbm.at[pl.ds(base_row, rows_per_w)])

    return pool


def _project(pooled, W, b):
    B, D = pooled.shape
    M = W.shape[0]
    BLK = 4096

    def mm(x_ref, w_ref, b_ref, o_ref):
        o_ref[...] = lax.dot_general(
            x_ref[...], w_ref[...], (((1,), (1,)), ((), ())),
            preferred_element_type=jnp.float32) + b_ref[...]

    return pl.pallas_call(
        mm,
        grid=(B // BLK,),
        in_specs=[
            pl.BlockSpec((BLK, D), lambda i: (i, 0)),
            pl.BlockSpec((M, D), lambda i: (0, 0)),
            pl.BlockSpec((1, M), lambda i: (0, 0)),
        ],
        out_specs=pl.BlockSpec((BLK, M), lambda i: (i, 0)),
        out_shape=jax.ShapeDtypeStruct((B, M), jnp.float32),
    )(pooled, W, b.reshape(1, M))


def kernel(token_ids, emb, W, b):
    B, Lseq = token_ids.shape
    idx_per_w = (B // _NW) * Lseq
    chunk = 2 * Lseq
    tok = token_ids.astype(jnp.int32).reshape(
        _NW, idx_per_w // chunk, chunk)
    pooled = _make_pool_kernel(B, Lseq, emb.shape[1])(tok, emb)
    out = _project(pooled, W, b)
    return out[:, None, :]


# speedup vs baseline: 1.0310x; 1.0049x over previous
import functools

import jax
import jax.numpy as jnp
from jax import lax
from jax.experimental import pallas as pl
from jax.experimental.pallas import tpu as pltpu
from jax.experimental.pallas import tpu_sc as plsc

_NUM_CORES = 2
_NUM_SUBCORES = 16
_NW = _NUM_CORES * _NUM_SUBCORES
_LANES = 16
_UNROLL = 5
_NBUF = 5


def _make_pool_kernel(B, Lseq, D):
    rows_per_w = B // _NW
    CR = 2
    chunk_len = CR * Lseq
    n_chunks = rows_per_w // CR
    nsub = D // _LANES
    scale = 1.0 / Lseq
    mesh = plsc.VectorSubcoreMesh(
        core_axis_name="c", subcore_axis_name="s",
        num_cores=_NUM_CORES, num_subcores=_NUM_SUBCORES)

    @functools.partial(
        pl.kernel,
        out_type=jax.ShapeDtypeStruct((B, D), jnp.float32),
        mesh=mesh,
        scratch_types=[
            pltpu.VMEM((n_chunks, chunk_len), jnp.int32),
            pltpu.VMEM((_NBUF, chunk_len, D), jnp.float32),
            pltpu.VMEM((rows_per_w, D), jnp.float32),
            [pltpu.SemaphoreType.DMA] * _NBUF,
        ],
    )
    def pool(tok_hbm, emb_hbm, out_hbm, idx_v, rows_v, pooled_v, gsem):
        wid = lax.axis_index("s") * _NUM_CORES + lax.axis_index("c")
        base_row = wid * rows_per_w
        pltpu.sync_copy(tok_hbm.at[wid], idx_v)

        def start(chunk, b):
            pltpu.async_copy(emb_hbm.at[idx_v.at[chunk]], rows_v.at[b],
                             gsem[b])

        def wait(chunk, b):
            pltpu.make_async_copy(
                emb_hbm.at[idx_v.at[chunk]], rows_v.at[b], gsem[b]).wait()

        def accumulate(chunk, b):
            for r in range(CR):
                def body(t, accs, r=r):
                    base = r * Lseq + t * _UNROLL
                    for u in range(_UNROLL):
                        accs = tuple(
                            accs[c] + rows_v[b, base + u,
                                             pl.ds(c * _LANES, _LANES)]
                            for c in range(nsub))
                    return accs
                accs = lax.fori_loop(
                    0, Lseq // _UNROLL, body,
                    tuple(jnp.zeros((_LANES,), jnp.float32)
                          for _ in range(nsub)))
                row = chunk * CR + r
                for c in range(nsub):
                    pooled_v[row, pl.ds(c * _LANES, _LANES)] = accs[c] * scale

        look = _NBUF - 1
        for p in range(look):
            start(p, p)

        def visit(k, b, cool):
            wait(k, b)
            if cool:
                start(k + look, (b + look) % _NBUF)
            accumulate(k, b)

        ngrp = (n_chunks - look) // _NBUF

        def grp(i, carry):
            k = _NBUF * i
            for q in range(_NBUF):
                visit(k + q, q, cool=True)
            return carry
        lax.fori_loop(0, ngrp, grp, 0)

        for kk in range(_NBUF * ngrp, n_chunks):
            visit(kk, kk % _NBUF, cool=kk + look < n_chunks)

        pltpu.sync_copy(pooled_v, out_hbm.at[pl.ds(base_row, rows_per_w)])

    return pool


def _project(pooled, W, b):
    B, D = pooled.shape
    M = W.shape[0]
    BLK = 4096

    def mm(x_ref, w_ref, b_ref, o_ref):
        o_ref[...] = lax.dot_general(
            x_ref[...], w_ref[...], (((1,), (1,)), ((), ())),
            preferred_element_type=jnp.float32) + b_ref[...]

    return pl.pallas_call(
        mm,
        grid=(B // BLK,),
        in_specs=[
            pl.BlockSpec((BLK, D), lambda i: (i, 0)),
            pl.BlockSpec((M, D), lambda i: (0, 0)),
            pl.BlockSpec((1, M), lambda i: (0, 0)),
        ],
        out_specs=pl.BlockSpec((BLK, M), lambda i: (i, 0)),
        out_shape=jax.ShapeDtypeStruct((B, M), jnp.float32),
    )(pooled, W, b.reshape(1, M))


def kernel(token_ids, emb, W, b):
    B, Lseq = token_ids.shape
    idx_per_w = (B // _NW) * Lseq
    chunk = 2 * Lseq
    tok = token_ids.astype(jnp.int32).reshape(
        _NW, idx_per_w // chunk, chunk)
    pooled = _make_pool_kernel(B, Lseq, emb.shape[1])(tok, emb)
    out = _project(pooled, W, b)
    return out[:, None, :]


# speedup vs baseline: 1.0343x; 1.0032x over previous
import functools

import jax
import jax.numpy as jnp
from jax import lax
from jax.experimental import pallas as pl
from jax.experimental.pallas import tpu as pltpu
from jax.experimental.pallas import tpu_sc as plsc

_NUM_CORES = 2
_NUM_SUBCORES = 16
_NW = _NUM_CORES * _NUM_SUBCORES
_LANES = 16
_UNROLL = 5
_NBUF = 5


def _make_pool_kernel(B, Lseq, D):
    rows_per_w = B // _NW
    CR = 2
    chunk_len = CR * Lseq
    n_chunks = rows_per_w // CR
    nsub = D // _LANES
    scale = 1.0 / Lseq
    mesh = plsc.VectorSubcoreMesh(
        core_axis_name="c", subcore_axis_name="s",
        num_cores=_NUM_CORES, num_subcores=_NUM_SUBCORES)

    @functools.partial(
        pl.kernel,
        out_type=jax.ShapeDtypeStruct((B, D), jnp.float32),
        mesh=mesh,
        scratch_types=[
            pltpu.VMEM((n_chunks, chunk_len), jnp.int32),
            pltpu.VMEM((_NBUF, chunk_len, D), jnp.float32),
            pltpu.VMEM((rows_per_w, D), jnp.float32),
            [pltpu.SemaphoreType.DMA] * _NBUF,
            pltpu.SemaphoreType.DMA,
        ],
    )
    def pool(tok_hbm, emb_hbm, out_hbm, idx_v, rows_v, pooled_v, gsem,
             isem):
        wid = lax.axis_index("s") * _NUM_CORES + lax.axis_index("c")
        base_row = wid * rows_per_w
        head = 8
        pltpu.sync_copy(tok_hbm.at[wid, pl.ds(0, head)],
                        idx_v.at[pl.ds(0, head)])
        idx_rest = pltpu.async_copy(
            tok_hbm.at[wid, pl.ds(head, n_chunks - head)],
            idx_v.at[pl.ds(head, n_chunks - head)], isem)

        def start(chunk, b):
            pltpu.async_copy(emb_hbm.at[idx_v.at[chunk]], rows_v.at[b],
                             gsem[b])

        def wait(chunk, b):
            pltpu.make_async_copy(
                emb_hbm.at[idx_v.at[chunk]], rows_v.at[b], gsem[b]).wait()

        def accumulate(chunk, b):
            for r in range(CR):
                def body(t, accs, r=r):
                    base = r * Lseq + t * _UNROLL
                    for u in range(_UNROLL):
                        accs = tuple(
                            accs[c] + rows_v[b, base + u,
                                             pl.ds(c * _LANES, _LANES)]
                            for c in range(nsub))
                    return accs
                accs = lax.fori_loop(
                    0, Lseq // _UNROLL, body,
                    tuple(jnp.zeros((_LANES,), jnp.float32)
                          for _ in range(nsub)))
                row = chunk * CR + r
                for c in range(nsub):
                    pooled_v[row, pl.ds(c * _LANES, _LANES)] = accs[c] * scale

        look = _NBUF - 1
        for p in range(look):
            start(p, p)
        idx_rest.wait()

        def visit(k, b, cool):
            wait(k, b)
            if cool:
                start(k + look, (b + look) % _NBUF)
            accumulate(k, b)

        ngrp = (n_chunks - look) // _NBUF

        def grp(i, carry):
            k = _NBUF * i
            for q in range(_NBUF):
                visit(k + q, q, cool=True)
            return carry
        lax.fori_loop(0, ngrp, grp, 0)

        for kk in range(_NBUF * ngrp, n_chunks):
            visit(kk, kk % _NBUF, cool=kk + look < n_chunks)

        pltpu.sync_copy(pooled_v, out_hbm.at[pl.ds(base_row, rows_per_w)])

    return pool


def _project(pooled, W, b):
    B, D = pooled.shape
    M = W.shape[0]
    BLK = 4096

    def mm(x_ref, w_ref, b_ref, o_ref):
        o_ref[...] = lax.dot_general(
            x_ref[...], w_ref[...], (((1,), (1,)), ((), ())),
            preferred_element_type=jnp.float32) + b_ref[...]

    return pl.pallas_call(
        mm,
        grid=(B // BLK,),
        in_specs=[
            pl.BlockSpec((BLK, D), lambda i: (i, 0)),
            pl.BlockSpec((M, D), lambda i: (0, 0)),
            pl.BlockSpec((1, M), lambda i: (0, 0)),
        ],
        out_specs=pl.BlockSpec((BLK, M), lambda i: (i, 0)),
        out_shape=jax.ShapeDtypeStruct((B, M), jnp.float32),
    )(pooled, W, b.reshape(1, M))


def kernel(token_ids, emb, W, b):
    B, Lseq = token_ids.shape
    idx_per_w = (B // _NW) * Lseq
    chunk = 2 * Lseq
    tok = token_ids.astype(jnp.int32).reshape(
        _NW, idx_per_w // chunk, chunk)
    pooled = _make_pool_kernel(B, Lseq, emb.shape[1])(tok, emb)
    out = _project(pooled, W, b)
    return out[:, None, :]
